# baseline (speedup 1.0000x reference)
import functools

import jax
import jax.numpy as jnp
from jax import lax
from jax.experimental import pallas as pl
from jax.experimental.pallas import tpu as pltpu
from jax.experimental.pallas import tpu_sc as plsc

N = 10000
E = 160000
D = 256
HALF = 128
G = 64

NS = 16
EPW = E // NS
CH = 128
NBUF = 3
NCHUNK = 81
EPAD = NCHUNK * CH
SUBROWS = 632
ACCROWS = NS * SUBROWS

BN = 1000
NB = N // BN


def _sc_edge_segsum(x_lo, x_hi, idx4):
    mesh = plsc.VectorSubcoreMesh(core_axis_name="c", subcore_axis_name="s")

    @functools.partial(
        pl.kernel,
        out_type=[
            jax.ShapeDtypeStruct((ACCROWS, HALF), jnp.float32),
            jax.ShapeDtypeStruct((ACCROWS, HALF), jnp.float32),
        ],
        mesh=mesh,
        scratch_types=(
            [pltpu.VMEM((2, CH), jnp.int32) for _ in range(NBUF)]
            + [pltpu.VMEM((CH, HALF), jnp.float32) for _ in range(NBUF)]
            + [pltpu.VMEM_SHARED((ACCROWS, HALF), jnp.float32)]
            + [pltpu.SemaphoreType.DMA for _ in range(3 * NBUF)]
        ),
    )
    def k(lo_hbm, hi_hbm, idx_hbm, out_lo, out_hi, *rest):
        idxb = rest[:NBUF]
        rows = rest[NBUF:2 * NBUF]
        acc = rest[2 * NBUF]
        sems = rest[2 * NBUF + 1:]
        isem = sems[:NBUF]
        gsem = sems[NBUF:2 * NBUF]
        ssem = sems[2 * NBUF:]
        c = lax.axis_index("c")
        s = lax.axis_index("s")

        zero16 = jnp.zeros((16,), jnp.float32)

        @pl.loop(0, 120)
        def _(i):
            @pl.loop(0, HALF, step=16)
            def _(j):
                rows[0][i, pl.ds(j, 16)] = zero16

        zbase = s * SUBROWS

        @pl.loop(0, 600, step=120)
        def _(r):
            pltpu.async_copy(rows[0].at[pl.ds(0, 120)],
                             acc.at[pl.ds(zbase + r, 120)], gsem[0])

        pltpu.async_copy(rows[0].at[pl.ds(0, 32)],
                         acc.at[pl.ds(zbase + 600, 32)], gsem[0])

        @pl.loop(0, 600, step=120)
        def _(r):
            pltpu.make_async_copy(rows[0].at[pl.ds(0, 120)],
                                  acc.at[pl.ds(0, 120)], gsem[0]).wait()

        pltpu.make_async_copy(rows[0].at[pl.ds(0, 32)],
                              acc.at[pl.ds(0, 32)], gsem[0]).wait()

        plsc.subcore_barrier()

        def start_i(j, b):
            pltpu.async_copy(idx_hbm.at[s, j], idxb[b], isem[b])

        def wait_i(b):
            pltpu.make_async_copy(idx_hbm.at[s, 0], idxb[b],
                                  isem[b]).wait()

        def start_g(b):
            @pl.when(c == 0)
            def _():
                pltpu.async_copy(lo_hbm.at[idxb[b].at[0]], rows[b],
                                 gsem[b])

            @pl.when(c == 1)
            def _():
                pltpu.async_copy(hi_hbm.at[idxb[b].at[0]], rows[b],
                                 gsem[b])

        def wait_g(b):
            pltpu.make_async_copy(lo_hbm.at[idxb[0].at[0]], rows[b],
                                  gsem[b]).wait()

        def start_s(b):
            pltpu.async_copy(rows[b], acc.at[idxb[b].at[1]], ssem[b],
                             add=True)

        def wait_s(b):
            pltpu.make_async_copy(rows[b], acc.at[idxb[0].at[1]],
                                  ssem[b]).wait()

        start_i(0, 0)
        start_i(1, 1)
        wait_i(0)
        start_g(0)

        @pl.loop(0, NCHUNK, step=NBUF)
        def _(j):
            for t in range(NBUF):
                jj = j + t
                b, b1, b2 = t, (t + 1) % NBUF, (t + 2) % NBUF

                @pl.when(jj >= 1)
                def _():
                    wait_s(b2)

                @pl.when(jj + 2 < NCHUNK)
                def _():
                    start_i(jj + 2, b2)

                @pl.when(jj + 1 < NCHUNK)
                def _():
                    wait_i(b1)
                    start_g(b1)

                wait_g(b)
                start_s(b)

        wait_s((NCHUNK - 1) % NBUF)

        plsc.subcore_barrier()

        def drain(out):
            @pl.loop(0, 600, step=120)
            def _(r):
                pltpu.async_copy(acc.at[pl.ds(zbase + r, 120)],
                                 out.at[pl.ds(zbase + r, 120)], gsem[0])

            pltpu.async_copy(acc.at[pl.ds(zbase + 600, 32)],
                             out.at[pl.ds(zbase + 600, 32)], gsem[0])

        @pl.when(c == 0)
        def _():
            drain(out_lo)

        @pl.when(c == 1)
        def _():
            drain(out_hi)

        @pl.loop(0, 600, step=120)
        def _(r):
            pltpu.make_async_copy(acc.at[pl.ds(0, 120)],
                                  out_lo.at[pl.ds(0, 120)], gsem[0]).wait()

        pltpu.make_async_copy(acc.at[pl.ds(0, 32)],
                              out_lo.at[pl.ds(0, 32)], gsem[0]).wait()

    return k(x_lo, x_hi, idx4)


def _pack_idx(src, dst):
    def pad(a, fill):
        a2 = a.reshape(NS, EPW)
        a2 = jnp.pad(a2, ((0, 0), (0, EPAD - EPW)), constant_values=fill)
        return a2.reshape(NS, NCHUNK, CH)
    return jnp.stack([pad(src, 0), pad(dst, N)], axis=2)


def _dot(a, b):
    return jnp.dot(a, b, preferred_element_type=jnp.float32,
                   precision=lax.Precision.DEFAULT)


def _mlp_core(sc_ref, hlo_in, hhi_in, alo, ahi, W1b, b1b, W2b, b2b, bb, po):
    h = jnp.concatenate([hlo_in[...], hhi_in[...]], axis=1)
    agg = jnp.concatenate([alo[...], ahi[...]], axis=1)
    z = sc_ref[0] * h + agg
    z = jnp.maximum(_dot(z, W1b[...]) + b1b[...], 0.0)
    z = _dot(z, W2b[...]) + b2b[...] + h
    seg = bb[0, 0, :]
    onehot = (seg[None, :] ==
              lax.broadcasted_iota(jnp.int32, (G, BN), 0)).astype(jnp.float32)
    contrib = _dot(onehot, z)

    @pl.when(pl.program_id(0) == 0)
    def _():
        po[...] = contrib

    @pl.when(pl.program_id(0) != 0)
    def _():
        po[...] = po[...] + contrib

    return z


_MLP_IN_SPECS = [
    pl.BlockSpec(memory_space=pltpu.SMEM),
    pl.BlockSpec((BN, HALF), lambda i: (i, 0)),
    pl.BlockSpec((BN, HALF), lambda i: (i, 0)),
    pl.BlockSpec((BN, HALF), lambda i: (i, 0)),
    pl.BlockSpec((BN, HALF), lambda i: (i, 0)),
    pl.BlockSpec((D, D), lambda i: (0, 0)),
    pl.BlockSpec((1, D), lambda i: (0, 0)),
    pl.BlockSpec((D, D), lambda i: (0, 0)),
    pl.BlockSpec((1, D), lambda i: (0, 0)),
    pl.BlockSpec((1, 1, BN), lambda i: (i, 0, 0)),
]


def _tc_gin_mlp(h_lo, h_hi, agg_lo, agg_hi, scale, W1, b1, W2, b2, batch3):
    def body(sc_ref, hlo_in, hhi_in, alo, ahi, W1b, b1b, W2b, b2b, bb,
             hlo, hhi, po):
        z = _mlp_core(sc_ref, hlo_in, hhi_in, alo, ahi,
                      W1b, b1b, W2b, b2b, bb, po)
        hlo[...] = z[:, :HALF]
        hhi[...] = z[:, HALF:]

    return pl.pallas_call(
        body,
        grid=(NB,),
        in_specs=_MLP_IN_SPECS,
        out_specs=[
            pl.BlockSpec((BN, HALF), lambda i: (i, 0)),
            pl.BlockSpec((BN, HALF), lambda i: (i, 0)),
            pl.BlockSpec((G, D), lambda i: (0, 0)),
        ],
        out_shape=[
            jax.ShapeDtypeStruct((N, HALF), jnp.float32),
            jax.ShapeDtypeStruct((N, HALF), jnp.float32),
            jax.ShapeDtypeStruct((G, D), jnp.float32),
        ],
    )(scale, h_lo, h_hi, agg_lo, agg_hi, W1, b1, W2, b2, batch3)


def _tc_gin_mlp_last(h_lo, h_hi, agg_lo, agg_hi, scale, W1, b1, W2, b2,
                     batch3, p0, p1, Wp, bp):
    def body(sc_ref, hlo_in, hhi_in, alo, ahi, W1b, b1b, W2b, b2b, bb,
             p0b, p1b, wpb, bpb, ho, po, ge):
        z = _mlp_core(sc_ref, hlo_in, hhi_in, alo, ahi,
                      W1b, b1b, W2b, b2b, bb, po)
        ho[...] = z

        @pl.when(pl.program_id(0) == NB - 1)
        def _():
            ge[...] = (_dot(p0b[...], wpb[0:D, :])
                       + _dot(p1b[...], wpb[D:2 * D, :])
                       + _dot(po[...], wpb[2 * D:, :]) + bpb[...])

    return pl.pallas_call(
        body,
        grid=(NB,),
        in_specs=_MLP_IN_SPECS + [
            pl.BlockSpec((G, D), lambda i: (0, 0)),
            pl.BlockSpec((G, D), lambda i: (0, 0)),
            pl.BlockSpec((3 * D, D), lambda i: (0, 0)),
            pl.BlockSpec((1, D), lambda i: (0, 0)),
        ],
        out_specs=[
            pl.BlockSpec((BN, D), lambda i: (i, 0)),
            pl.BlockSpec((G, D), lambda i: (0, 0)),
            pl.BlockSpec((G, D), lambda i: (0, 0)),
        ],
        out_shape=[
            jax.ShapeDtypeStruct((N, D), jnp.float32),
            jax.ShapeDtypeStruct((G, D), jnp.float32),
            jax.ShapeDtypeStruct((G, D), jnp.float32),
        ],
    )(scale, h_lo, h_hi, agg_lo, agg_hi, W1, b1, W2, b2, batch3,
      p0, p1, Wp, bp)


def kernel(x, edge_index, batch,
           eps0, l0_W1, l0_b1, l0_W2, l0_b2,
           eps1, l1_W1, l1_b1, l1_W2, l1_b2,
           eps2, l2_W1, l2_b1, l2_W2, l2_b2,
           pool_W, pool_b):
    idx4 = _pack_idx(edge_index[0].astype(jnp.int32),
                     edge_index[1].astype(jnp.int32))
    batch3 = batch.astype(jnp.int32).reshape(NB, 1, BN)

    layer_params = [
        (eps0, l0_W1, l0_b1, l0_W2, l0_b2),
        (eps1, l1_W1, l1_b1, l1_W2, l1_b2),
        (eps2, l2_W1, l2_b1, l2_W2, l2_b2),
    ]

    h_lo = x[:, :HALF]
    h_hi = x[:, HALF:]
    pooled = []
    for li, (eps, W1, b1, W2, b2) in enumerate(layer_params):
        agg_lo, agg_hi = _sc_edge_segsum(h_lo, h_hi, idx4)
        scale = (1.0 + eps).reshape(1).astype(jnp.float32)
        args = (h_lo, h_hi, agg_lo, agg_hi, scale, W1, b1.reshape(1, D),
                W2, b2.reshape(1, D), batch3)
        if li < 2:
            h_lo, h_hi, po = _tc_gin_mlp(*args)
            pooled.append(po)
        else:
            h, po, ge = _tc_gin_mlp_last(
                *args, pooled[0], pooled[1], pool_W, pool_b.reshape(1, D))

    return (h, ge)


# speedup vs baseline: 1.9899x; 1.9899x over previous
import functools

import jax
import jax.numpy as jnp
from jax import lax
from jax.experimental import pallas as pl
from jax.experimental.pallas import tpu as pltpu
from jax.experimental.pallas import tpu_sc as plsc

N = 10000
E = 160000
D = 256
HALF = 128
G = 64

NS = 16
EPW = E // NS
CH = 120
NBUF = 3
NCHUNK = 84
EPAD = NCHUNK * CH
SUBROWS = 640
ACCROWS = NS * SUBROWS
ZCH = 80

BN = 1000
NB = N // BN


def _sc_edge_segsum(x_lo, x_hi, src4, dst4):
    mesh = plsc.VectorSubcoreMesh(core_axis_name="c", subcore_axis_name="s")

    @functools.partial(
        pl.kernel,
        out_type=[
            jax.ShapeDtypeStruct((ACCROWS, HALF), jnp.float32),
            jax.ShapeDtypeStruct((ACCROWS, HALF), jnp.float32),
        ],
        mesh=mesh,
        scratch_types=(
            [pltpu.VMEM((CH,), jnp.int32) for _ in range(NBUF)]
            + [pltpu.VMEM((CH,), jnp.int32) for _ in range(NBUF)]
            + [pltpu.VMEM((CH, HALF), jnp.float32) for _ in range(NBUF)]
            + [pltpu.VMEM_SHARED((ACCROWS, HALF), jnp.float32)]
            + [pltpu.SemaphoreType.DMA for _ in range(3 * NBUF)]
        ),
    )
    def k(lo_hbm, hi_hbm, src_hbm, dst_hbm, out_lo, out_hi, *rest):
        srcb = rest[:NBUF]
        dstb = rest[NBUF:2 * NBUF]
        rows = rest[2 * NBUF:3 * NBUF]
        acc = rest[3 * NBUF]
        isem = rest[3 * NBUF + 1:3 * NBUF + 1 + NBUF]
        gsem = rest[3 * NBUF + 1 + NBUF:3 * NBUF + 1 + 2 * NBUF]
        ssem = rest[3 * NBUF + 1 + 2 * NBUF:]
        c = lax.axis_index("c")
        s = lax.axis_index("s")

        zero16 = jnp.zeros((16,), jnp.float32)

        @pl.loop(0, ZCH)
        def _(i):
            @pl.loop(0, HALF, step=16)
            def _(j):
                rows[0][i, pl.ds(j, 16)] = zero16

        @pl.loop(0, SUBROWS, step=ZCH)
        def _(r):
            pltpu.async_copy(rows[0].at[pl.ds(0, ZCH)],
                             acc.at[pl.ds(s * SUBROWS + r, ZCH)], gsem[0])

        @pl.loop(0, SUBROWS, step=ZCH)
        def _(r):
            pltpu.make_async_copy(rows[0].at[pl.ds(0, ZCH)],
                                  acc.at[pl.ds(0, ZCH)], gsem[0]).wait()

        plsc.subcore_barrier()

        def start_i(j, b):
            pltpu.async_copy(src_hbm.at[s, j, 0], srcb[b], isem[b])
            pltpu.async_copy(dst_hbm.at[s, j, 0], dstb[b], isem[b])

        def wait_i(b):
            pltpu.make_async_copy(src_hbm.at[s, 0, 0], srcb[b],
                                  isem[b]).wait()
            pltpu.make_async_copy(dst_hbm.at[s, 0, 0], dstb[b],
                                  isem[b]).wait()

        def start_g(b):
            @pl.when(c == 0)
            def _():
                pltpu.async_copy(lo_hbm.at[srcb[b]], rows[b], gsem[b])

            @pl.when(c == 1)
            def _():
                pltpu.async_copy(hi_hbm.at[srcb[b]], rows[b], gsem[b])

        def wait_g(b):
            pltpu.make_async_copy(lo_hbm.at[srcb[b]], rows[b],
                                  gsem[b]).wait()

        def start_s(b):
            pltpu.async_copy(rows[b], acc.at[dstb[b]], ssem[b], add=True)

        def wait_s(b):
            pltpu.make_async_copy(rows[b], acc.at[dstb[b]], ssem[b]).wait()

        start_i(0, 0)
        start_i(1, 1)
        wait_i(0)
        start_g(0)

        @pl.loop(0, NCHUNK, step=NBUF)
        def _(j):
            for t in range(NBUF):
                jj = j + t
                b, b1, b2 = t, (t + 1) % NBUF, (t + 2) % NBUF

                @pl.when(jj >= 1)
                def _():
                    wait_s(b2)

                @pl.when(jj + 2 < NCHUNK)
                def _():
                    start_i(jj + 2, b2)

                @pl.when(jj + 1 < NCHUNK)
                def _():
                    wait_i(b1)
                    start_g(b1)

                wait_g(b)
                start_s(b)

        wait_s((NCHUNK - 1) % NBUF)

        plsc.subcore_barrier()

        base = s * SUBROWS

        @pl.loop(0, SUBROWS, step=ZCH)
        def _(r):
            @pl.when(c == 0)
            def _():
                pltpu.async_copy(acc.at[pl.ds(base + r, ZCH)],
                                 out_lo.at[pl.ds(base + r, ZCH)], gsem[0])

            @pl.when(c == 1)
            def _():
                pltpu.async_copy(acc.at[pl.ds(base + r, ZCH)],
                                 out_hi.at[pl.ds(base + r, ZCH)], gsem[0])

        @pl.loop(0, SUBROWS, step=ZCH)
        def _(r):
            pltpu.make_async_copy(acc.at[pl.ds(0, ZCH)],
                                  out_lo.at[pl.ds(0, ZCH)], gsem[0]).wait()

    return k(x_lo, x_hi, src4, dst4)


def _pad_idx(a, fill):
    a2 = a.reshape(NS, EPW)
    a2 = jnp.pad(a2, ((0, 0), (0, EPAD - EPW)), constant_values=fill)
    return a2.reshape(NS, NCHUNK, 1, CH)


def _dot(a, b):
    return jnp.dot(a, b, preferred_element_type=jnp.float32,
                   precision=lax.Precision.DEFAULT)


def _mlp_core(sc_ref, hlo_in, hhi_in, alo, ahi, W1b, b1b, W2b, b2b, bb, po):
    h = jnp.concatenate([hlo_in[...], hhi_in[...]], axis=1)
    agg = jnp.concatenate([alo[...], ahi[...]], axis=1)
    z = sc_ref[0] * h + agg
    z = jnp.maximum(_dot(z, W1b[...]) + b1b[...], 0.0)
    z = _dot(z, W2b[...]) + b2b[...] + h
    seg = bb[0, 0, :]
    onehot = (seg[None, :] ==
              lax.broadcasted_iota(jnp.int32, (G, BN), 0)).astype(jnp.float32)
    contrib = _dot(onehot, z)

    @pl.when(pl.program_id(0) == 0)
    def _():
        po[...] = contrib

    @pl.when(pl.program_id(0) != 0)
    def _():
        po[...] = po[...] + contrib

    return z


_MLP_IN_SPECS = [
    pl.BlockSpec(memory_space=pltpu.SMEM),
    pl.BlockSpec((BN, HALF), lambda i: (i, 0)),
    pl.BlockSpec((BN, HALF), lambda i: (i, 0)),
    pl.BlockSpec((BN, HALF), lambda i: (i, 0)),
    pl.BlockSpec((BN, HALF), lambda i: (i, 0)),
    pl.BlockSpec((D, D), lambda i: (0, 0)),
    pl.BlockSpec((1, D), lambda i: (0, 0)),
    pl.BlockSpec((D, D), lambda i: (0, 0)),
    pl.BlockSpec((1, D), lambda i: (0, 0)),
    pl.BlockSpec((1, 1, BN), lambda i: (i, 0, 0)),
]


def _tc_gin_mlp(h_lo, h_hi, agg_lo, agg_hi, scale, W1, b1, W2, b2, batch3):
    def body(sc_ref, hlo_in, hhi_in, alo, ahi, W1b, b1b, W2b, b2b, bb,
             hlo, hhi, po):
        z = _mlp_core(sc_ref, hlo_in, hhi_in, alo, ahi,
                      W1b, b1b, W2b, b2b, bb, po)
        hlo[...] = z[:, :HALF]
        hhi[...] = z[:, HALF:]

    return pl.pallas_call(
        body,
        grid=(NB,),
        in_specs=_MLP_IN_SPECS,
        out_specs=[
            pl.BlockSpec((BN, HALF), lambda i: (i, 0)),
            pl.BlockSpec((BN, HALF), lambda i: (i, 0)),
            pl.BlockSpec((G, D), lambda i: (0, 0)),
        ],
        out_shape=[
            jax.ShapeDtypeStruct((N, HALF), jnp.float32),
            jax.ShapeDtypeStruct((N, HALF), jnp.float32),
            jax.ShapeDtypeStruct((G, D), jnp.float32),
        ],
    )(scale, h_lo, h_hi, agg_lo, agg_hi, W1, b1, W2, b2, batch3)


def _tc_gin_mlp_last(h_lo, h_hi, agg_lo, agg_hi, scale, W1, b1, W2, b2,
                     batch3, p0, p1, Wp, bp):
    def body(sc_ref, hlo_in, hhi_in, alo, ahi, W1b, b1b, W2b, b2b, bb,
             p0b, p1b, wpb, bpb, ho, po, ge):
        z = _mlp_core(sc_ref, hlo_in, hhi_in, alo, ahi,
                      W1b, b1b, W2b, b2b, bb, po)
        ho[...] = z

        @pl.when(pl.program_id(0) == NB - 1)
        def _():
            ge[...] = (_dot(p0b[...], wpb[0:D, :])
                       + _dot(p1b[...], wpb[D:2 * D, :])
                       + _dot(po[...], wpb[2 * D:, :]) + bpb[...])

    return pl.pallas_call(
        body,
        grid=(NB,),
        in_specs=_MLP_IN_SPECS + [
            pl.BlockSpec((G, D), lambda i: (0, 0)),
            pl.BlockSpec((G, D), lambda i: (0, 0)),
            pl.BlockSpec((3 * D, D), lambda i: (0, 0)),
            pl.BlockSpec((1, D), lambda i: (0, 0)),
        ],
        out_specs=[
            pl.BlockSpec((BN, D), lambda i: (i, 0)),
            pl.BlockSpec((G, D), lambda i: (0, 0)),
            pl.BlockSpec((G, D), lambda i: (0, 0)),
        ],
        out_shape=[
            jax.ShapeDtypeStruct((N, D), jnp.float32),
            jax.ShapeDtypeStruct((G, D), jnp.float32),
            jax.ShapeDtypeStruct((G, D), jnp.float32),
        ],
    )(scale, h_lo, h_hi, agg_lo, agg_hi, W1, b1, W2, b2, batch3,
      p0, p1, Wp, bp)


def kernel(x, edge_index, batch,
           eps0, l0_W1, l0_b1, l0_W2, l0_b2,
           eps1, l1_W1, l1_b1, l1_W2, l1_b2,
           eps2, l2_W1, l2_b1, l2_W2, l2_b2,
           pool_W, pool_b):
    src4 = _pad_idx(edge_index[0].astype(jnp.int32), 0)
    dst4 = _pad_idx(edge_index[1].astype(jnp.int32), N)
    batch3 = batch.astype(jnp.int32).reshape(NB, 1, BN)

    layer_params = [
        (eps0, l0_W1, l0_b1, l0_W2, l0_b2),
        (eps1, l1_W1, l1_b1, l1_W2, l1_b2),
        (eps2, l2_W1, l2_b1, l2_W2, l2_b2),
    ]

    h_lo = x[:, :HALF]
    h_hi = x[:, HALF:]
    pooled = []
    for li, (eps, W1, b1, W2, b2) in enumerate(layer_params):
        agg_lo, agg_hi = _sc_edge_segsum(h_lo, h_hi, src4, dst4)
        scale = (1.0 + eps).reshape(1).astype(jnp.float32)
        args = (h_lo, h_hi, agg_lo, agg_hi, scale, W1, b1.reshape(1, D),
                W2, b2.reshape(1, D), batch3)
        if li < 2:
            h_lo, h_hi, po = _tc_gin_mlp(*args)
            pooled.append(po)
        else:
            h, po, ge = _tc_gin_mlp_last(
                *args, pooled[0], pooled[1], pool_W, pool_b.reshape(1, D))

    return (h, ge)
